# baseline (speedup 1.0000x reference)
import functools

import jax
import jax.numpy as jnp
from jax.experimental import pallas as pl
from jax.experimental.pallas import tpu as pltpu

_NUM_EXPERTS = 8
_TOP_K = 2
_D_IN = 768
_D_OUT = 768
_TOKENS = 8192

_BLOCK_T = 2048


def _moe_block_kernel(x_ref, wet_ref, be_ref, wg_ref, bg_ref, out_ref):
    x = x_ref[...]

    logits = jax.lax.dot_general(x, wg_ref[...], (((1,), (1,)), ((), ())),
                                 preferred_element_type=jnp.float32,
                                 precision=jax.lax.Precision.DEFAULT)
    logits = logits + bg_ref[...]

    e_iota = jax.lax.broadcasted_iota(jnp.int32, logits.shape, 1)
    neg = jnp.float32(-1e30)

    m1 = jnp.max(logits, axis=-1, keepdims=True)
    i1 = jnp.min(jnp.where(logits == m1, e_iota, _NUM_EXPERTS),
                 axis=-1, keepdims=True)
    mask1 = e_iota == i1
    l2 = jnp.where(mask1, neg, logits)
    m2 = jnp.max(l2, axis=-1, keepdims=True)
    i2 = jnp.min(jnp.where(l2 == m2, e_iota, _NUM_EXPERTS),
                 axis=-1, keepdims=True)
    mask2 = e_iota == i2

    e2 = jnp.exp(m2 - m1)
    inv = 1.0 / (1.0 + e2)
    w = jnp.where(mask1, inv, 0.0) + jnp.where(mask2, e2 * inv, 0.0)

    acc = jnp.dot(w, be_ref[...], preferred_element_type=jnp.float32,
                  precision=jax.lax.Precision.DEFAULT)
    dn = (((1,), (1,)), ((), ()))
    for e in range(_NUM_EXPERTS):
        y = jax.lax.dot_general(x, wet_ref[e], dn,
                                preferred_element_type=jnp.float32,
                                precision=jax.lax.Precision.DEFAULT)
        acc = acc + w[:, e:e + 1] * y
    out_ref[...] = acc


@jax.jit
def kernel(x, We, be, Wg, bg):
    bg2 = bg.reshape(1, _NUM_EXPERTS)

    grid = (_TOKENS // _BLOCK_T,)
    return pl.pallas_call(
        _moe_block_kernel,
        grid=grid,
        in_specs=[
            pl.BlockSpec((_BLOCK_T, _D_IN), lambda i: (i, 0)),
            pl.BlockSpec((_NUM_EXPERTS, _D_OUT, _D_IN), lambda i: (0, 0, 0)),
            pl.BlockSpec((_NUM_EXPERTS, _D_OUT), lambda i: (0, 0)),
            pl.BlockSpec((_NUM_EXPERTS, _D_IN), lambda i: (0, 0)),
            pl.BlockSpec((1, _NUM_EXPERTS), lambda i: (0, 0)),
        ],
        out_specs=pl.BlockSpec((_BLOCK_T, _D_OUT), lambda i: (i, 0)),
        out_shape=jax.ShapeDtypeStruct((_TOKENS, _D_OUT), jnp.float32),
    )(x, We, be, Wg, bg2)


# speedup vs baseline: 1.0549x; 1.0549x over previous
import functools

import jax
import jax.numpy as jnp
from jax.experimental import pallas as pl
from jax.experimental.pallas import tpu as pltpu

_NUM_EXPERTS = 8
_TOP_K = 2
_D_IN = 768
_D_OUT = 768
_TOKENS = 8192

_BLOCK_T = 1024


def _moe_block_kernel(x_ref, wet_ref, be_ref, wg_ref, bg_ref, out_ref):
    x = x_ref[...]

    logits = jax.lax.dot_general(x, wg_ref[...], (((1,), (1,)), ((), ())),
                                 preferred_element_type=jnp.float32,
                                 precision=jax.lax.Precision.DEFAULT)
    logits = logits + bg_ref[...]

    e_iota = jax.lax.broadcasted_iota(jnp.int32, logits.shape, 1)
    neg = jnp.float32(-1e30)

    m1 = jnp.max(logits, axis=-1, keepdims=True)
    i1 = jnp.min(jnp.where(logits == m1, e_iota, _NUM_EXPERTS),
                 axis=-1, keepdims=True)
    mask1 = e_iota == i1
    l2 = jnp.where(mask1, neg, logits)
    m2 = jnp.max(l2, axis=-1, keepdims=True)
    i2 = jnp.min(jnp.where(l2 == m2, e_iota, _NUM_EXPERTS),
                 axis=-1, keepdims=True)
    mask2 = e_iota == i2

    e2 = jnp.exp(m2 - m1)
    inv = 1.0 / (1.0 + e2)
    w = jnp.where(mask1, inv, 0.0) + jnp.where(mask2, e2 * inv, 0.0)

    acc = jnp.dot(w, be_ref[...], preferred_element_type=jnp.float32,
                  precision=jax.lax.Precision.DEFAULT)
    dn = (((1,), (1,)), ((), ()))
    for e in range(_NUM_EXPERTS):
        y = jax.lax.dot_general(x, wet_ref[e], dn,
                                preferred_element_type=jnp.float32,
                                precision=jax.lax.Precision.DEFAULT)
        acc = acc + w[:, e:e + 1] * y
    out_ref[...] = acc


@jax.jit
def kernel(x, We, be, Wg, bg):
    bg2 = bg.reshape(1, _NUM_EXPERTS)

    grid = (_TOKENS // _BLOCK_T,)
    return pl.pallas_call(
        _moe_block_kernel,
        grid=grid,
        in_specs=[
            pl.BlockSpec((_BLOCK_T, _D_IN), lambda i: (i, 0)),
            pl.BlockSpec((_NUM_EXPERTS, _D_OUT, _D_IN), lambda i: (0, 0, 0)),
            pl.BlockSpec((_NUM_EXPERTS, _D_OUT), lambda i: (0, 0)),
            pl.BlockSpec((_NUM_EXPERTS, _D_IN), lambda i: (0, 0)),
            pl.BlockSpec((1, _NUM_EXPERTS), lambda i: (0, 0)),
        ],
        out_specs=pl.BlockSpec((_BLOCK_T, _D_OUT), lambda i: (i, 0)),
        out_shape=jax.ShapeDtypeStruct((_TOKENS, _D_OUT), jnp.float32),
    )(x, We, be, Wg, bg2)


# speedup vs baseline: 1.0552x; 1.0003x over previous
import jax
import jax.numpy as jnp
from jax.experimental import pallas as pl
from jax.experimental.pallas import tpu as pltpu

_NUM_EXPERTS = 8
_TOP_K = 2
_D_IN = 768
_D_OUT = 768
_TOKENS = 8192

_BLOCK_T = 1024


def _moe_block_kernel(x_ref, wet_ref, be_ref, wg_ref, bg_ref, out_ref):
    x = x_ref[...]

    logits = jax.lax.dot_general(x, wg_ref[...], (((1,), (1,)), ((), ())),
                                 preferred_element_type=jnp.float32,
                                 precision=jax.lax.Precision.DEFAULT)
    logits = logits + bg_ref[...]

    e_iota = jax.lax.broadcasted_iota(jnp.int32, logits.shape, 1)
    neg = jnp.float32(-1e30)

    m1 = jnp.max(logits, axis=-1, keepdims=True)
    i1 = jnp.min(jnp.where(logits == m1, e_iota, _NUM_EXPERTS),
                 axis=-1, keepdims=True)
    mask1 = e_iota == i1
    l2 = jnp.where(mask1, neg, logits)
    m2 = jnp.max(l2, axis=-1, keepdims=True)
    i2 = jnp.min(jnp.where(l2 == m2, e_iota, _NUM_EXPERTS),
                 axis=-1, keepdims=True)
    mask2 = e_iota == i2

    e2 = jnp.exp(m2 - m1)
    inv = 1.0 / (1.0 + e2)
    w = jnp.where(mask1, inv, 0.0) + jnp.where(mask2, e2 * inv, 0.0)

    acc = jnp.dot(w, be_ref[...], preferred_element_type=jnp.float32,
                  precision=jax.lax.Precision.DEFAULT)
    dn = (((1,), (1,)), ((), ()))
    for e in range(_NUM_EXPERTS):
        y = jax.lax.dot_general(x, wet_ref[e], dn,
                                preferred_element_type=jnp.float32,
                                precision=jax.lax.Precision.DEFAULT)
        acc = acc + w[:, e:e + 1] * y
    out_ref[...] = acc


@jax.jit
def kernel(x, We, be, Wg, bg):
    bg2 = bg.reshape(1, _NUM_EXPERTS)

    grid = (_TOKENS // _BLOCK_T,)
    return pl.pallas_call(
        _moe_block_kernel,
        grid=grid,
        in_specs=[
            pl.BlockSpec((_BLOCK_T, _D_IN), lambda i: (i, 0)),
            pl.BlockSpec((_NUM_EXPERTS, _D_OUT, _D_IN), lambda i: (0, 0, 0)),
            pl.BlockSpec((_NUM_EXPERTS, _D_OUT), lambda i: (0, 0)),
            pl.BlockSpec((_NUM_EXPERTS, _D_IN), lambda i: (0, 0)),
            pl.BlockSpec((1, _NUM_EXPERTS), lambda i: (0, 0)),
        ],
        out_specs=pl.BlockSpec((_BLOCK_T, _D_OUT), lambda i: (i, 0)),
        out_shape=jax.ShapeDtypeStruct((_TOKENS, _D_OUT), jnp.float32),
    )(x, We, be, Wg, bg2)


# speedup vs baseline: 1.0564x; 1.0011x over previous
import jax
import jax.numpy as jnp
from jax.experimental import pallas as pl

_NUM_EXPERTS = 8
_TOP_K = 2
_D_IN = 768
_D_OUT = 768
_TOKENS = 8192

_BLOCK_T = 1024


def _moe_block_kernel(x_ref, wet_ref, be_ref, wg_ref, bg_ref, out_ref):
    x = x_ref[...]

    logits = jax.lax.dot_general(x, wg_ref[...], (((1,), (1,)), ((), ())),
                                 preferred_element_type=jnp.float32,
                                 precision=jax.lax.Precision.DEFAULT)
    logits = logits + bg_ref[...]

    e_iota = jax.lax.broadcasted_iota(jnp.int32, logits.shape, 1)
    neg = jnp.float32(-1e30)

    m1 = jnp.max(logits, axis=-1, keepdims=True)
    i1 = jnp.min(jnp.where(logits == m1, e_iota, _NUM_EXPERTS),
                 axis=-1, keepdims=True)
    mask1 = e_iota == i1
    l2 = jnp.where(mask1, neg, logits)
    m2 = jnp.max(l2, axis=-1, keepdims=True)
    i2 = jnp.min(jnp.where(l2 == m2, e_iota, _NUM_EXPERTS),
                 axis=-1, keepdims=True)
    mask2 = e_iota == i2

    e2 = jnp.exp(m2 - m1)
    inv = 1.0 / (1.0 + e2)
    w = jnp.where(mask1, inv, 0.0) + jnp.where(mask2, e2 * inv, 0.0)

    acc = jnp.dot(w, be_ref[...], preferred_element_type=jnp.float32,
                  precision=jax.lax.Precision.DEFAULT)
    dn = (((1,), (1,)), ((), ()))
    for e in range(_NUM_EXPERTS):
        y = jax.lax.dot_general(x, wet_ref[e], dn,
                                preferred_element_type=jnp.float32,
                                precision=jax.lax.Precision.DEFAULT)
        acc = acc + w[:, e:e + 1] * y
    out_ref[...] = acc


@jax.jit
def kernel(x, We, be, Wg, bg):
    bg2 = bg.reshape(1, _NUM_EXPERTS)

    grid = (_TOKENS // _BLOCK_T,)
    return pl.pallas_call(
        _moe_block_kernel,
        grid=grid,
        in_specs=[
            pl.BlockSpec((_BLOCK_T, _D_IN), lambda i: (i, 0)),
            pl.BlockSpec((_NUM_EXPERTS, _D_OUT, _D_IN), lambda i: (0, 0, 0)),
            pl.BlockSpec((_NUM_EXPERTS, _D_OUT), lambda i: (0, 0)),
            pl.BlockSpec((_NUM_EXPERTS, _D_IN), lambda i: (0, 0)),
            pl.BlockSpec((1, _NUM_EXPERTS), lambda i: (0, 0)),
        ],
        out_specs=pl.BlockSpec((_BLOCK_T, _D_OUT), lambda i: (i, 0)),
        out_shape=jax.ShapeDtypeStruct((_TOKENS, _D_OUT), jnp.float32),
    )(x, We, be, Wg, bg2)
